# baseline (speedup 1.0000x reference)
import jax
import jax.numpy as jnp
from jax import lax
from jax.experimental import pallas as pl
from jax.experimental.pallas import tpu as pltpu

_TOPK = 100
_THRESH = -1000000000.0
_NEG = -1000000000.0


def _halve_max(cur, w):
    while w > 1:
        w //= 2
        cur = jnp.maximum(cur[:, :w, :], cur[:, w:, :])
    return cur


def _sizes(R):
    P = R // 2
    G1 = 16 if P % 16 == 0 else P
    N1 = P // G1
    G0 = 16 if (N1 % 16 == 0 and N1 >= 16) else N1
    N0 = N1 // G0
    C = 512 if R % 512 == 0 else R
    NC = R // C
    return P, G1, N1, G0, N0, C, NC


def _make_body(BS, R, W, H, topk):
    P, G1, N1, G0, N0, C, NC = _sizes(R)
    PC = C // 2
    LC = PC // G1

    def body(x_ref, top_ref, bot_ref, score_ref, view_ref, row_ref, col_ref,
             *scratch):
        vs = scratch[0::4]
        pas = scratch[1::4]
        l1s = scratch[2::4]
        l0s = scratch[3::4]
        step = pl.program_id(0)
        b = step // NC
        k = step % NC
        ninf = jnp.float32(-jnp.inf)

        @pl.when(step < BS * NC)
        def nms_phase():
            nrow2 = jnp.full((2, W), ninf, jnp.float32)
            top2 = jnp.where(k > 0, top_ref[0, 6:8, :], nrow2)
            bot2 = jnp.where(k < NC - 1, bot_ref[0, 0:2, :], nrow2)
            xa = jnp.concatenate([top2, x_ref[0], bot2], 0)
            ncol1 = jnp.full((C + 4, 1), ninf, jnp.float32)
            ncol2 = jnp.full((C + 4, 2), ninf, jnp.float32)
            h = jnp.maximum(
                jnp.maximum(xa, jnp.concatenate([xa[:, 1:], ncol1], 1)),
                jnp.concatenate([ncol1, xa[:, :-1]], 1),
            )
            h = jnp.maximum(
                h,
                jnp.maximum(
                    jnp.concatenate([xa[:, 2:], ncol2], 1),
                    jnp.concatenate([ncol2, xa[:, :-2]], 1),
                ),
            )
            vv = jnp.maximum(
                jnp.maximum(h[2: C + 2, :], h[: C, :]),
                jnp.maximum(h[1: C + 1, :], h[3: C + 3, :]),
            )
            vv = jnp.maximum(vv, h[4: C + 4, :])
            xc = xa[2: C + 2, :]
            m = jnp.where(xc == vv, xc, jnp.float32(_NEG))
            m2 = m.reshape(PC, 2, W)
            r0 = m2[:, 0, :]
            r1 = m2[:, 1, :]
            win = jnp.maximum(r0, r1)
            pa = jnp.where(r1 > r0, jnp.float32(1),
                           jnp.where(r1 == r0, jnp.float32(2), jnp.float32(0)))
            pab = pa.astype(jnp.bfloat16)
            l1c = _halve_max(win.reshape(LC, G1, W), G1).reshape(LC, W)
            for bb in range(BS):
                @pl.when(b == bb)
                def store_chunk(bb=bb):
                    vs[bb][pl.ds(k * PC, PC), :] = win
                    pas[bb][pl.ds(k * PC, PC), :] = pab
                    l1s[bb][pl.ds(k * LC, LC), :] = l1c

                    @pl.when(k == NC - 1)
                    def build_l0():
                        l1 = l1s[bb][...]
                        l0s[bb][...] = _halve_max(
                            l1.reshape(N0, G0, W), G0
                        ).reshape(N0, W)

        @pl.when(step == BS * NC)
        def select_phase():
            score_ref[...] = jnp.zeros((BS, 128), jnp.float32)
            view_ref[...] = jnp.zeros((BS, 128), jnp.float32)
            row_ref[...] = jnp.zeros((BS, 128), jnp.float32)
            col_ref[...] = jnp.zeros((BS, 128), jnp.float32)

            lane128 = lax.broadcasted_iota(jnp.int32, (1, 128), 1)
            iota0 = lax.broadcasted_iota(jnp.int32, (N0, W), 0)
            iotag0 = lax.broadcasted_iota(jnp.int32, (G0, W), 0)
            iotag1 = lax.broadcasted_iota(jnp.int32, (G1, W), 0)
            iotac = lax.broadcasted_iota(jnp.int32, (1, W), 1)
            iota16r = lax.broadcasted_iota(jnp.int32, (16, W), 0)
            iota16c = lax.broadcasted_iota(jnp.int32, (16, W), 1)

            def iter_body(i, carry):
                lm = lane128 == i
                for bb in range(BS):
                    v_ref, pa_ref = vs[bb], pas[bb]
                    l1_ref, l0_ref = l1s[bb], l0s[bb]
                    l0b = l0_ref[...]
                    vb = jnp.max(l0b)
                    s0 = jnp.min(jnp.where(l0b == vb, iota0, N0))
                    l1g = l1_ref[pl.ds(s0 * G0, G0), :]
                    s1 = s0 * G0 + jnp.min(jnp.where(l1g == vb, iotag0, G0))
                    vg = v_ref[pl.ds(s1 * G1, G1), :]
                    s2 = s1 * G1 + jnp.min(jnp.where(vg == vb, iotag1, G1))
                    vrow = v_ref[pl.ds(s2, 1), :]
                    base = pl.multiple_of((s2 // 16) * 16, 16)
                    rmask = iota16r == (s2 % 16)
                    pa16 = pa_ref[pl.ds(base, 16), :].astype(jnp.float32)
                    parow = jnp.max(
                        jnp.where(rmask, pa16, 0.0), axis=0, keepdims=True)
                    eq = vrow == vb
                    peff = (parow == 1.0).astype(jnp.int32)
                    minp = jnp.min(jnp.where(eq, peff, 2))
                    sel = eq & (peff == minp)
                    c = jnp.min(jnp.where(sel, iotac, W))
                    lc = iotac == c
                    both = jnp.sum(jnp.where(lc, parow, 0.0)) == 2.0
                    r = 2 * s2 + minp
                    score_ref[pl.ds(bb, 1), :] = jnp.where(
                        lm, vb, score_ref[pl.ds(bb, 1), :])
                    view_ref[pl.ds(bb, 1), :] = jnp.where(
                        lm, (r // H).astype(jnp.float32), view_ref[pl.ds(bb, 1), :])
                    row_ref[pl.ds(bb, 1), :] = jnp.where(
                        lm, (r % H).astype(jnp.float32), row_ref[pl.ds(bb, 1), :])
                    col_ref[pl.ds(bb, 1), :] = jnp.where(
                        lm, c.astype(jnp.float32), col_ref[pl.ds(bb, 1), :])
                    v_ref[pl.ds(s2, 1), :] = jnp.where(
                        lc & jnp.logical_not(both), ninf, vrow)
                    pa_ref[pl.ds(base, 16), :] = jnp.where(
                        rmask & (iota16c == c), jnp.float32(1), pa16
                    ).astype(jnp.bfloat16)
                    l1_ref[pl.ds(s1, 1), :] = jnp.max(
                        v_ref[pl.ds(s1 * G1, G1), :], axis=0, keepdims=True)
                    l0_ref[pl.ds(s0, 1), :] = jnp.max(
                        l1_ref[pl.ds(s0 * G0, G0), :], axis=0, keepdims=True)
                return carry

            lax.fori_loop(0, topk, iter_body, 0)

    return body


def kernel(heatmap_logits):
    bs, num_img, _, H, W = heatmap_logits.shape
    R = num_img * H
    hm = heatmap_logits.reshape(bs, R, W)
    topk = min(_TOPK, R * W)
    P, _, N1, _, N0, C, NC = _sizes(R)
    C8 = C // 8
    R8 = R // 8
    S = bs * NC

    def ix_main(s):
        bb = jnp.minimum(s // NC, bs - 1)
        return (bb, jnp.where(s < S, s % NC, 0), 0)

    def ix_top(s):
        bb = jnp.minimum(s // NC, bs - 1)
        return (bb, jnp.maximum((s % NC) * C8 - 1, 0), 0)

    def ix_bot(s):
        bb = jnp.minimum(s // NC, bs - 1)
        return (bb, jnp.minimum((s % NC) * C8 + C8, R8 - 1), 0)

    scratch = []
    for _ in range(bs):
        scratch += [
            pltpu.VMEM((P, W), jnp.float32),
            pltpu.VMEM((P, W), jnp.bfloat16),
            pltpu.VMEM((N1, W), jnp.float32),
            pltpu.VMEM((N0, W), jnp.float32),
        ]

    body = _make_body(bs, R, W, H, topk)
    outs = pl.pallas_call(
        body,
        grid=(S + 1,),
        in_specs=[
            pl.BlockSpec((1, C, W), ix_main),
            pl.BlockSpec((1, 8, W), ix_top),
            pl.BlockSpec((1, 8, W), ix_bot),
        ],
        out_specs=[pl.BlockSpec((bs, 128), lambda s: (0, 0)) for _ in range(4)],
        out_shape=[jax.ShapeDtypeStruct((bs, 128), jnp.float32) for _ in range(4)],
        scratch_shapes=scratch,
    )(hm, hm, hm)
    scores128, views128, rows128, cols128 = outs
    scores = scores128[:, :topk]
    peak_positions = jnp.stack(
        [views128[:, :topk], rows128[:, :topk], cols128[:, :topk]], axis=-1
    )
    peak_mask = scores > _THRESH
    return peak_positions, scores, peak_mask


# speedup vs baseline: 1.7082x; 1.7082x over previous
import jax
import jax.numpy as jnp
from jax import lax
from jax.experimental import pallas as pl
from jax.experimental.pallas import tpu as pltpu

_TOPK = 100
_THRESH = -1000000000.0
_NEG = -1000000000.0


def _halve_max(cur, w):
    while w > 1:
        w //= 2
        cur = jnp.maximum(cur[:, :w, :], cur[:, w:, :])
    return cur


def _halve_or(cur, w):
    while w > 1:
        w //= 2
        cur = cur[:, :w, :] | cur[:, w:, :]
    return cur


def _sizes(R):
    P = R // 2
    G1 = 16 if P % 16 == 0 else P
    N1 = P // G1
    C = 512 if R % 512 == 0 else R
    NC = R // C
    return P, G1, N1, C, NC


def _make_body(BS, R, W, H, topk):
    P, G1, N1, C, NC = _sizes(R)
    PC = C // 2
    LC = PC // G1

    def body(x_ref, top_ref, bot_ref, score_ref, view_ref, row_ref, col_ref,
             *scratch):
        vs = scratch[0::3]
        pas = scratch[1::3]
        l1s = scratch[2::3]
        step = pl.program_id(0)
        b = step // NC
        k = step % NC
        ninf = jnp.float32(-jnp.inf)

        @pl.when(step < BS * NC)
        def nms_phase():
            nrow2 = jnp.full((2, W), ninf, jnp.float32)
            top2 = jnp.where(k > 0, top_ref[0, 6:8, :], nrow2)
            bot2 = jnp.where(k < NC - 1, bot_ref[0, 0:2, :], nrow2)
            xa = jnp.concatenate([top2, x_ref[0], bot2], 0)
            ncol1 = jnp.full((C + 4, 1), ninf, jnp.float32)
            ncol2 = jnp.full((C + 4, 2), ninf, jnp.float32)
            h = jnp.maximum(
                jnp.maximum(xa, jnp.concatenate([xa[:, 1:], ncol1], 1)),
                jnp.concatenate([ncol1, xa[:, :-1]], 1),
            )
            h = jnp.maximum(
                h,
                jnp.maximum(
                    jnp.concatenate([xa[:, 2:], ncol2], 1),
                    jnp.concatenate([ncol2, xa[:, :-2]], 1),
                ),
            )
            vv = jnp.maximum(
                jnp.maximum(h[2: C + 2, :], h[: C, :]),
                jnp.maximum(h[1: C + 1, :], h[3: C + 3, :]),
            )
            vv = jnp.maximum(vv, h[4: C + 4, :])
            xc = xa[2: C + 2, :]
            m = jnp.where(xc == vv, xc, jnp.float32(_NEG))
            m2 = m.reshape(PC, 2, W)
            r0 = m2[:, 0, :]
            r1 = m2[:, 1, :]
            win = jnp.maximum(r0, r1)
            pa = jnp.where(r1 > r0, 1, jnp.where(r1 == r0, 2, 0))
            rr = lax.broadcasted_iota(jnp.int32, (PC, W), 0) % 16
            packed = _halve_or((pa << (2 * rr)).reshape(PC // 16, 16, W), 16)
            packed = packed.reshape(PC // 16, W)
            l1c = _halve_max(win.reshape(LC, G1, W), G1).reshape(LC, W)
            for bb in range(BS):
                @pl.when(b == bb)
                def store_chunk(bb=bb):
                    vs[bb][pl.ds(k * PC, PC), :] = win
                    pas[bb][pl.ds(k * (PC // 16), PC // 16), :] = packed
                    l1s[bb][pl.ds(k * LC, LC), :] = l1c

        @pl.when(step == BS * NC)
        def select_phase():
            lane128 = lax.broadcasted_iota(jnp.int32, (1, 128), 1)
            iotan1 = lax.broadcasted_iota(jnp.int32, (N1, W), 0)
            iotag1 = lax.broadcasted_iota(jnp.int32, (G1, W), 0)
            iotac = lax.broadcasted_iota(jnp.int32, (1, W), 1)
            brow = lax.broadcasted_iota(jnp.int32, (BS, 128), 0)

            def iter_body(i, carry):
                sc, vw, rw, cw = carry
                lmr = lane128 == i
                for bb in range(BS):
                    v_ref, pa_ref, l1_ref = vs[bb], pas[bb], l1s[bb]
                    l1 = l1_ref[...]
                    vb = jnp.max(
                        jnp.max(l1, axis=0, keepdims=True), axis=1, keepdims=True)
                    s1 = jnp.min(jnp.where(l1 == vb, iotan1, N1))
                    vg = v_ref[pl.ds(s1 * G1, G1), :]
                    s2 = s1 * G1 + jnp.min(jnp.where(vg == vb, iotag1, G1))
                    vrow = v_ref[pl.ds(s2, 1), :]
                    prow = pa_ref[pl.ds(s2 // 16, 1), :]
                    sh = 2 * (s2 % 16)
                    pav = (prow >> sh) & 3
                    peff = (pav == 1).astype(jnp.int32)
                    eq = vrow == vb
                    minp = jnp.min(jnp.where(eq, peff, 2), axis=1, keepdims=True)
                    sel = eq & (peff == minp)
                    c = jnp.min(jnp.where(sel, iotac, W))
                    lc = iotac == c
                    v_ref[pl.ds(s2, 1), :] = jnp.where(
                        lc & (pav != 2), ninf, vrow)
                    pa_ref[pl.ds(s2 // 16, 1), :] = jnp.where(
                        lc, (prow & ~(3 << sh)) | (1 << sh), prow)
                    l1_ref[pl.ds(s1, 1), :] = jnp.max(
                        v_ref[pl.ds(s1 * G1, G1), :], axis=0, keepdims=True)
                    upd = (brow == bb) & lmr
                    sc = jnp.where(upd, vb, sc)
                    vw = jnp.where(upd, jnp.float32(s2 // (H // 2)), vw)
                    rw = jnp.where(
                        upd, jnp.float32((2 * s2) % H) + minp.astype(jnp.float32),
                        rw)
                    cw = jnp.where(upd, c.astype(jnp.float32), cw)
                return sc, vw, rw, cw

            zero = jnp.zeros((BS, 128), jnp.float32)
            sc, vw, rw, cw = lax.fori_loop(
                0, topk, iter_body, (zero, zero, zero, zero))
            score_ref[...] = sc
            view_ref[...] = vw
            row_ref[...] = rw
            col_ref[...] = cw

    return body


def kernel(heatmap_logits):
    bs, num_img, _, H, W = heatmap_logits.shape
    R = num_img * H
    hm = heatmap_logits.reshape(bs, R, W)
    topk = min(_TOPK, R * W)
    P, _, N1, C, NC = _sizes(R)
    C8 = C // 8
    R8 = R // 8
    S = bs * NC

    def ix_main(s):
        bb = jnp.minimum(s // NC, bs - 1)
        return (bb, jnp.where(s < S, s % NC, 0), 0)

    def ix_top(s):
        bb = jnp.minimum(s // NC, bs - 1)
        return (bb, jnp.maximum((s % NC) * C8 - 1, 0), 0)

    def ix_bot(s):
        bb = jnp.minimum(s // NC, bs - 1)
        return (bb, jnp.minimum((s % NC) * C8 + C8, R8 - 1), 0)

    scratch = []
    for _ in range(bs):
        scratch += [
            pltpu.VMEM((P, W), jnp.float32),
            pltpu.VMEM((P // 16, W), jnp.int32),
            pltpu.VMEM((N1, W), jnp.float32),
        ]

    body = _make_body(bs, R, W, H, topk)
    outs = pl.pallas_call(
        body,
        grid=(S + 1,),
        in_specs=[
            pl.BlockSpec((1, C, W), ix_main),
            pl.BlockSpec((1, 8, W), ix_top),
            pl.BlockSpec((1, 8, W), ix_bot),
        ],
        out_specs=[pl.BlockSpec((bs, 128), lambda s: (0, 0)) for _ in range(4)],
        out_shape=[jax.ShapeDtypeStruct((bs, 128), jnp.float32) for _ in range(4)],
        scratch_shapes=scratch,
    )(hm, hm, hm)
    scores128, views128, rows128, cols128 = outs
    scores = scores128[:, :topk]
    peak_positions = jnp.stack(
        [views128[:, :topk], rows128[:, :topk], cols128[:, :topk]], axis=-1
    )
    peak_mask = scores > _THRESH
    return peak_positions, scores, peak_mask


# speedup vs baseline: 4.0007x; 2.3421x over previous
import jax
import jax.numpy as jnp
from jax import lax
from jax.experimental import pallas as pl
from jax.experimental.pallas import tpu as pltpu

_TOPK = 100
_THRESH = -1000000000.0
_NEG = -1000000000.0


def _halve_max(cur, w):
    while w > 1:
        w //= 2
        cur = jnp.maximum(cur[:, :w, :], cur[:, w:, :])
    return cur


def _halve_or(cur, w):
    while w > 1:
        w //= 2
        cur = cur[:, :w, :] | cur[:, w:, :]
    return cur


def _sizes(R):
    P = R // 2
    G1 = 16 if P % 16 == 0 else P
    N1 = P // G1
    C = 512 if R % 512 == 0 else R
    NC = R // C
    return P, G1, N1, C, NC


def _make_body(BS, R, W, H, topk):
    P, G1, N1, C, NC = _sizes(R)
    PC = C // 2
    LC = PC // G1

    def body(x_ref, top_ref, bot_ref, score_ref, view_ref, row_ref, col_ref,
             *scratch):
        vs = scratch[0::3]
        pas = scratch[1::3]
        l1s = scratch[2::3]
        step = pl.program_id(0)
        b = step // NC
        k = step % NC
        ninf = jnp.float32(-jnp.inf)

        @pl.when(step < BS * NC)
        def nms_phase():
            nrow2 = jnp.full((2, W), ninf, jnp.float32)
            top2 = jnp.where(k > 0, top_ref[0, 6:8, :], nrow2)
            bot2 = jnp.where(k < NC - 1, bot_ref[0, 0:2, :], nrow2)
            xa = jnp.concatenate([top2, x_ref[0], bot2], 0)
            ncol1 = jnp.full((C + 4, 1), ninf, jnp.float32)
            ncol2 = jnp.full((C + 4, 2), ninf, jnp.float32)
            h = jnp.maximum(
                jnp.maximum(xa, jnp.concatenate([xa[:, 1:], ncol1], 1)),
                jnp.concatenate([ncol1, xa[:, :-1]], 1),
            )
            h = jnp.maximum(
                h,
                jnp.maximum(
                    jnp.concatenate([xa[:, 2:], ncol2], 1),
                    jnp.concatenate([ncol2, xa[:, :-2]], 1),
                ),
            )
            vv = jnp.maximum(
                jnp.maximum(h[2: C + 2, :], h[: C, :]),
                jnp.maximum(h[1: C + 1, :], h[3: C + 3, :]),
            )
            vv = jnp.maximum(vv, h[4: C + 4, :])
            xc = xa[2: C + 2, :]
            m = jnp.where(xc == vv, xc, jnp.float32(_NEG))
            m2 = m.reshape(PC, 2, W)
            r0 = m2[:, 0, :]
            r1 = m2[:, 1, :]
            win = jnp.maximum(r0, r1)
            pa = jnp.where(r1 > r0, 1, jnp.where(r1 == r0, 2, 0))
            rr = lax.broadcasted_iota(jnp.int32, (PC, W), 0) % 16
            packed = _halve_or((pa << (2 * rr)).reshape(PC // 16, 16, W), 16)
            packed = packed.reshape(PC // 16, W)
            l1c = _halve_max(win.reshape(LC, G1, W), G1).reshape(LC, W)
            for bb in range(BS):
                @pl.when(b == bb)
                def store_chunk(bb=bb):
                    vs[bb][pl.ds(k * PC, PC), :] = win
                    pas[bb][pl.ds(k * (PC // 16), PC // 16), :] = packed
                    l1s[bb][pl.ds(k * LC, LC), :] = l1c

        @pl.when(step == BS * NC)
        def select_phase():
            lane128 = lax.broadcasted_iota(jnp.int32, (1, 128), 1)
            iotan1 = lax.broadcasted_iota(jnp.int32, (N1, W), 0)
            iotag1 = lax.broadcasted_iota(jnp.int32, (G1, W), 0)
            iotac = lax.broadcasted_iota(jnp.int32, (1, W), 1)
            brow = lax.broadcasted_iota(jnp.int32, (BS, 128), 0)

            def iter_body(i, carry):
                sc, vw, rw, cw = carry
                lmr = lane128 == i
                for bb in range(BS):
                    v_ref, pa_ref, l1_ref = vs[bb], pas[bb], l1s[bb]
                    l1 = l1_ref[...]
                    vb = jnp.max(
                        jnp.max(l1, axis=0, keepdims=True), axis=1, keepdims=True)
                    s1 = jnp.min(jnp.where(l1 == vb, iotan1, N1))
                    vg = v_ref[pl.ds(s1 * G1, G1), :]
                    prow = pa_ref[pl.ds(s1, 1), :]
                    eq16 = vg == vb
                    s2l = jnp.min(
                        jnp.min(jnp.where(eq16, iotag1, G1), axis=0,
                                keepdims=True), axis=1, keepdims=True)
                    rm16 = iotag1 == s2l
                    vrow = jnp.max(
                        jnp.where(rm16, vg, ninf), axis=0, keepdims=True)
                    shv = 2 * s2l
                    pav = (prow >> shv) & 3
                    peff = (pav == 1).astype(jnp.int32)
                    eq = vrow == vb
                    minp = jnp.min(jnp.where(eq, peff, 2), axis=1, keepdims=True)
                    sel = eq & (peff == minp)
                    c = jnp.min(jnp.where(sel, iotac, W), axis=1, keepdims=True)
                    lc = iotac == c
                    vg_new = jnp.where(rm16 & lc & (pav != 2), ninf, vg)
                    v_ref[pl.ds(s1 * G1, G1), :] = vg_new
                    pa_ref[pl.ds(s1, 1), :] = jnp.where(
                        lc, (prow & ~(3 << shv)) | (1 << shv), prow)
                    l1_ref[pl.ds(s1, 1), :] = jnp.max(
                        vg_new, axis=0, keepdims=True)
                    upd = (brow == bb) & lmr
                    s2v = s1 * G1 + s2l
                    sc = jnp.where(upd, vb, sc)
                    vw = jnp.where(upd, (s2v // (H // 2)).astype(jnp.float32), vw)
                    rw = jnp.where(
                        upd, ((2 * s2v) % H + minp).astype(jnp.float32), rw)
                    cw = jnp.where(upd, c.astype(jnp.float32), cw)
                return sc, vw, rw, cw

            zero = jnp.zeros((BS, 128), jnp.float32)
            sc, vw, rw, cw = lax.fori_loop(
                0, topk, iter_body, (zero, zero, zero, zero))
            score_ref[...] = sc
            view_ref[...] = vw
            row_ref[...] = rw
            col_ref[...] = cw

    return body


def kernel(heatmap_logits):
    bs, num_img, _, H, W = heatmap_logits.shape
    R = num_img * H
    hm = heatmap_logits.reshape(bs, R, W)
    topk = min(_TOPK, R * W)
    P, _, N1, C, NC = _sizes(R)
    C8 = C // 8
    R8 = R // 8
    S = bs * NC

    def ix_main(s):
        bb = jnp.minimum(s // NC, bs - 1)
        return (bb, jnp.where(s < S, s % NC, 0), 0)

    def ix_top(s):
        bb = jnp.minimum(s // NC, bs - 1)
        return (bb, jnp.maximum((s % NC) * C8 - 1, 0), 0)

    def ix_bot(s):
        bb = jnp.minimum(s // NC, bs - 1)
        return (bb, jnp.minimum((s % NC) * C8 + C8, R8 - 1), 0)

    scratch = []
    for _ in range(bs):
        scratch += [
            pltpu.VMEM((P, W), jnp.float32),
            pltpu.VMEM((P // 16, W), jnp.int32),
            pltpu.VMEM((N1, W), jnp.float32),
        ]

    body = _make_body(bs, R, W, H, topk)
    outs = pl.pallas_call(
        body,
        grid=(S + 1,),
        in_specs=[
            pl.BlockSpec((1, C, W), ix_main),
            pl.BlockSpec((1, 8, W), ix_top),
            pl.BlockSpec((1, 8, W), ix_bot),
        ],
        out_specs=[pl.BlockSpec((bs, 128), lambda s: (0, 0)) for _ in range(4)],
        out_shape=[jax.ShapeDtypeStruct((bs, 128), jnp.float32) for _ in range(4)],
        scratch_shapes=scratch,
    )(hm, hm, hm)
    scores128, views128, rows128, cols128 = outs
    scores = scores128[:, :topk]
    peak_positions = jnp.stack(
        [views128[:, :topk], rows128[:, :topk], cols128[:, :topk]], axis=-1
    )
    peak_mask = scores > _THRESH
    return peak_positions, scores, peak_mask
